# baseline (speedup 1.0000x reference)
import jax
import jax.numpy as jnp
from jax.experimental import pallas as pl

_K = 25
_PAD = (_K - 1) // 2


def _decomp_body(x_ref, res_ref, mm_ref):
    x = x_ref[0]
    t = x.shape[0]
    front = jnp.broadcast_to(x[0:1], (_PAD, x.shape[1]))
    back = jnp.broadcast_to(x[t - 1:t], (_PAD, x.shape[1]))
    xp = jnp.concatenate([front, x, back], axis=0)
    a2 = xp[:-1] + xp[1:]
    a4 = a2[:-2] + a2[2:]
    a8 = a4[:-4] + a4[4:]
    a16 = a8[:-8] + a8[8:]
    a24 = a16[0:t] + a8[16:16 + t]
    s25 = a24 + xp[24:24 + t]
    mm = s25 * (1.0 / _K)
    res_ref[0] = x - mm
    mm_ref[0] = mm


def kernel(x):
    b, t, c = x.shape
    out = jax.ShapeDtypeStruct((b, t, c), x.dtype)
    cb = c // 2
    grid = (b, 2)
    spec = pl.BlockSpec((1, t, cb), lambda i, j: (i, 0, j))
    res, mm = pl.pallas_call(
        _decomp_body,
        grid=grid,
        in_specs=[spec],
        out_specs=(spec, spec),
        out_shape=(out, out),
    )(x)
    return (res, mm)


# speedup vs baseline: 1.1420x; 1.1420x over previous
import jax
import jax.numpy as jnp
from jax.experimental import pallas as pl

_K = 25
_PAD = (_K - 1) // 2


def _decomp_body(x_ref, res_ref, mm_ref):
    for n in range(x_ref.shape[0]):
        _decomp_one(n, x_ref, res_ref, mm_ref)


def _decomp_one(n, x_ref, res_ref, mm_ref):
    x = x_ref[n]
    t = x.shape[0]
    front = jnp.broadcast_to(x[0:1], (_PAD, x.shape[1]))
    back = jnp.broadcast_to(x[t - 1:t], (_PAD, x.shape[1]))
    xp = jnp.concatenate([front, x, back], axis=0)
    a2 = xp[:-1] + xp[1:]
    a4 = a2[:-2] + a2[2:]
    a8 = a4[:-4] + a4[4:]
    a16 = a8[:-8] + a8[8:]
    a24 = a16[0:t] + a8[16:16 + t]
    s25 = a24 + xp[24:24 + t]
    mm = s25 * (1.0 / _K)
    res_ref[n] = x - mm
    mm_ref[n] = mm


def kernel(x):
    b, t, c = x.shape
    out = jax.ShapeDtypeStruct((b, t, c), x.dtype)
    bb = 2
    grid = (b // bb,)
    spec = pl.BlockSpec((bb, t, c), lambda i: (i, 0, 0))
    res, mm = pl.pallas_call(
        _decomp_body,
        grid=grid,
        in_specs=[spec],
        out_specs=(spec, spec),
        out_shape=(out, out),
    )(x)
    return (res, mm)


# speedup vs baseline: 1.1427x; 1.0006x over previous
import jax
import jax.numpy as jnp
from jax.experimental import pallas as pl
from jax.experimental.pallas import tpu as pltpu

_K = 25
_PAD = (_K - 1) // 2
_FRONT = 16


def _decomp_body(x_ref, res_ref, mm_ref, xp_ref):
    for n in range(x_ref.shape[0]):
        _decomp_one(n, x_ref, res_ref, mm_ref, xp_ref)


def _decomp_one(n, x_ref, res_ref, mm_ref, xp_ref):
    x = x_ref[n]
    t, c = x.shape
    xp_ref[0:_FRONT] = jnp.broadcast_to(x[0:1], (_FRONT, c))
    xp_ref[_FRONT:_FRONT + t] = x
    xp_ref[_FRONT + t:] = jnp.broadcast_to(x[t - 1:t], (_FRONT, c))
    y = xp_ref[...]
    c2 = y[:-1] + y[1:]
    c4 = c2[:-2] + c2[2:]
    c8 = c4[:-4] + c4[4:]
    c16 = c8[:-8] + c8[8:]
    c24 = c16[0:t + 8] + c8[16:16 + t + 8]
    c25 = c24 + y[24:24 + t + 8]
    mm = c25[4:4 + t] * (1.0 / _K)
    res_ref[n] = x - mm
    mm_ref[n] = mm


def kernel(x):
    b, t, c = x.shape
    out = jax.ShapeDtypeStruct((b, t, c), x.dtype)
    bb = 2
    grid = (b // bb,)
    spec = pl.BlockSpec((bb, t, c), lambda i: (i, 0, 0))
    res, mm = pl.pallas_call(
        _decomp_body,
        grid=grid,
        in_specs=[spec],
        out_specs=(spec, spec),
        out_shape=(out, out),
        scratch_shapes=[pltpu.VMEM((t + 2 * _FRONT, c), x.dtype)],
    )(x)
    return (res, mm)
